# baseline (speedup 1.0000x reference)
import functools

import jax
import jax.numpy as jnp
from jax import lax
from jax.experimental import pallas as pl
from jax.experimental.pallas import tpu as pltpu


def _round_up(x, m):
    return ((x + m - 1) // m) * m


def _lstm_kernel(x_ref, wih_f_ref, whh_f_ref, b_f_ref,
                 wih_b_ref, whh_b_ref, b_b_ref, e_ref, xpf, xpb,
                 *, seq_len, batch, hidden):
    T, Bp, H = seq_len, batch, hidden

    x = x_ref[...]
    xpf[...] = (jnp.dot(x, wih_f_ref[...].astype(jnp.bfloat16),
                        preferred_element_type=jnp.float32) + b_f_ref[...])
    xpb[...] = (jnp.dot(x, wih_b_ref[...].astype(jnp.bfloat16),
                        preferred_element_type=jnp.float32) + b_b_ref[...])
    whh_f = whh_f_ref[...].astype(jnp.bfloat16)
    whh_b = whh_b_ref[...].astype(jnp.bfloat16)

    def cell(pre, c):
        i = jax.nn.sigmoid(pre[:, 0:H])
        f = jax.nn.sigmoid(pre[:, H:2 * H])
        g = jnp.tanh(pre[:, 2 * H:3 * H])
        o = jax.nn.sigmoid(pre[:, 3 * H:4 * H])
        c = f * c + i * g
        return o * jnp.tanh(c), c

    HB = Bp // 2
    z = jnp.zeros((HB, H), jnp.float32)
    hf, cf = [z, z], [z, z]
    hb, cb = [z, z], [z, z]
    for t in range(T):
        rf = t * Bp
        rb = (T - 1 - t) * Bp
        for k in (0, 1):
            pre = xpf[pl.ds(rf + k * HB, HB), :] + jnp.dot(
                hf[k].astype(jnp.bfloat16), whh_f,
                preferred_element_type=jnp.float32)
            hf[k], cf[k] = cell(pre, cf[k])
            e_ref[pl.ds(rf + k * HB, HB), 0:H] = hf[k].astype(jnp.bfloat16)
        for k in (0, 1):
            pre = xpb[pl.ds(rb + k * HB, HB), :] + jnp.dot(
                hb[k].astype(jnp.bfloat16), whh_b,
                preferred_element_type=jnp.float32)
            hb[k], cb[k] = cell(pre, cb[k])
            e_ref[pl.ds(rb + k * HB, HB), H:2 * H] = hb[k].astype(jnp.bfloat16)


def _head_kernel(e_ref, gum_ref, wb_ref, bb_ref, wd_ref, il_ref, lg_ref,
                 *, inv_temp, n_gumbel, gp):
    il = (jnp.dot(e_ref[...], wb_ref[...].astype(jnp.bfloat16),
                  preferred_element_type=jnp.float32) + bb_ref[...])
    il_ref[...] = il
    y = (il + gum_ref[...]) * inv_temp
    if n_gumbel < gp:
        lane = lax.broadcasted_iota(jnp.int32, y.shape, 1)
        y = jnp.where(lane < n_gumbel, y, jnp.float32(-1e30))
    m = jnp.max(y, axis=-1, keepdims=True)
    e = jnp.exp(y - m)
    s = jnp.sum(e, axis=-1, keepdims=True)
    enc = e * pl.reciprocal(s, approx=True)
    lg_ref[...] = jnp.dot(enc.astype(jnp.bfloat16),
                          wd_ref[...].astype(jnp.bfloat16),
                          preferred_element_type=jnp.float32)


def kernel(x, wih_f, whh_f, b_f, wih_b, whh_b, b_b, wb, bias_b, wd,
           gumbel_noise):
    B, F, T = x.shape
    H = whh_f.shape[0]
    G = wb.shape[-1]
    C = wd.shape[-1]
    Bp = _round_up(max(B, 8), 8)
    Gp = _round_up(max(G, 128), 128)
    Cp = _round_up(max(C, 128), 128)
    TBp = T * Bp

    x_tbf = jnp.transpose(x.astype(jnp.bfloat16), (2, 0, 1))
    x_tbf = jnp.pad(x_tbf, ((0, 0), (0, Bp - B), (0, 0)))
    x_2d = x_tbf.reshape(TBp, F)

    lstm = functools.partial(_lstm_kernel, seq_len=T, batch=Bp, hidden=H)
    full2 = lambda i: (0, 0)
    embed_tm = pl.pallas_call(
        lstm,
        grid=(1,),
        out_shape=jax.ShapeDtypeStruct((TBp, 2 * H), jnp.bfloat16),
        in_specs=[
            pl.BlockSpec((TBp, F), full2),
            pl.BlockSpec((F, 4 * H), full2),
            pl.BlockSpec((H, 4 * H), full2),
            pl.BlockSpec((1, 4 * H), full2),
            pl.BlockSpec((F, 4 * H), full2),
            pl.BlockSpec((H, 4 * H), full2),
            pl.BlockSpec((1, 4 * H), full2),
        ],
        out_specs=pl.BlockSpec((TBp, 2 * H), full2),
        scratch_shapes=[pltpu.VMEM((TBp, 4 * H), jnp.float32),
                        pltpu.VMEM((TBp, 4 * H), jnp.float32)],
        compiler_params=pltpu.CompilerParams(
            dimension_semantics=("arbitrary",)),
    )(x_2d, wih_f, whh_f, b_f, wih_b, whh_b, b_b)

    e_bm = jnp.transpose(embed_tm.reshape(T, Bp, 2 * H),
                         (1, 0, 2)).reshape(Bp * T, 2 * H)

    gum_2d = gumbel_noise.reshape(B * T, G)
    gum_2d = jnp.pad(gum_2d, ((0, (Bp - B) * T), (0, Gp - G)))

    wb_p = jnp.pad(wb, ((0, 0), (0, Gp - G)))
    bb_p = jnp.pad(bias_b, ((0, 0), (0, Gp - G)))
    wd_p = jnp.pad(wd, ((0, Gp - G), (0, Cp - C)))

    NBLK = 2
    R = TBp // NBLK
    row_map = lambda j: (j, 0)
    wmap = lambda j: (0, 0)
    head = functools.partial(_head_kernel, inv_temp=1.0, n_gumbel=G, gp=Gp)
    il2, lg2 = pl.pallas_call(
        head,
        grid=(NBLK,),
        out_shape=(jax.ShapeDtypeStruct((TBp, Gp), jnp.float32),
                   jax.ShapeDtypeStruct((TBp, Cp), jnp.float32)),
        in_specs=[
            pl.BlockSpec((R, 2 * H), row_map),
            pl.BlockSpec((R, Gp), row_map),
            pl.BlockSpec((2 * H, Gp), wmap),
            pl.BlockSpec((1, Gp), wmap),
            pl.BlockSpec((Gp, Cp), wmap),
        ],
        out_specs=(pl.BlockSpec((R, Gp), row_map),
                   pl.BlockSpec((R, Cp), row_map)),
        compiler_params=pltpu.CompilerParams(
            dimension_semantics=("arbitrary",)),
    )(e_bm, gum_2d, wb_p, bb_p, wd_p)

    in_logit = il2.reshape(Bp, T, Gp)[:B, :, :G]
    logit = lg2.reshape(Bp, T, Cp)[:B, :, :C]
    return in_logit, logit


# speedup vs baseline: 1.3456x; 1.3456x over previous
import functools

import jax
import jax.numpy as jnp
from jax import lax
from jax.experimental import pallas as pl
from jax.experimental.pallas import tpu as pltpu


def _round_up(x, m):
    return ((x + m - 1) // m) * m


def _lstm_kernel(x_ref, wih_f_ref, whh_f_ref, b_f_ref,
                 wih_b_ref, whh_b_ref, b_b_ref, e_ref, xpf, xpb,
                 *, seq_len, batch, hidden):
    T, Bp, H = seq_len, batch, hidden

    x = x_ref[...]
    xpf[...] = (jnp.dot(x, wih_f_ref[...].astype(jnp.bfloat16),
                        preferred_element_type=jnp.float32) + b_f_ref[...])
    xpb[...] = (jnp.dot(x, wih_b_ref[...].astype(jnp.bfloat16),
                        preferred_element_type=jnp.float32) + b_b_ref[...])
    whh_f = whh_f_ref[...].astype(jnp.bfloat16)
    whh_b = whh_b_ref[...].astype(jnp.bfloat16)

    def cell(pre, c):
        i = jax.nn.sigmoid(pre[:, 0:H])
        f = jax.nn.sigmoid(pre[:, H:2 * H])
        g = jnp.tanh(pre[:, 2 * H:3 * H])
        o = jax.nn.sigmoid(pre[:, 3 * H:4 * H])
        c = f * c + i * g
        return o * jnp.tanh(c), c

    z = jnp.zeros((Bp, H), jnp.float32)
    hf, cf, hb, cb = z, z, z, z
    for t in range(T):
        rf = t * Bp
        rb = (T - 1 - t) * Bp
        pre_f = xpf[pl.ds(rf, Bp), :] + jnp.dot(
            hf.astype(jnp.bfloat16), whh_f, preferred_element_type=jnp.float32)
        hf, cf = cell(pre_f, cf)
        e_ref[pl.ds(rf, Bp), 0:H] = hf.astype(jnp.bfloat16)
        pre_b = xpb[pl.ds(rb, Bp), :] + jnp.dot(
            hb.astype(jnp.bfloat16), whh_b, preferred_element_type=jnp.float32)
        hb, cb = cell(pre_b, cb)
        e_ref[pl.ds(rb, Bp), H:2 * H] = hb.astype(jnp.bfloat16)


def _head_kernel(e_ref, gum_ref, wb_ref, bb_ref, wd_ref, il_ref, lg_ref,
                 *, inv_temp, n_gumbel, gp):
    T, BT, HH = e_ref.shape
    e = jnp.transpose(e_ref[...], (1, 0, 2)).reshape(BT * T, HH)
    il = (jnp.dot(e, wb_ref[...].astype(jnp.bfloat16),
                  preferred_element_type=jnp.float32) + bb_ref[...])
    il_ref[...] = il
    y = (il + gum_ref[...]) * inv_temp
    if n_gumbel < gp:
        lane = lax.broadcasted_iota(jnp.int32, y.shape, 1)
        y = jnp.where(lane < n_gumbel, y, jnp.float32(-1e30))
    m = jnp.max(y, axis=-1, keepdims=True)
    e = jnp.exp(y - m)
    s = jnp.sum(e, axis=-1, keepdims=True)
    enc = e * pl.reciprocal(s, approx=True)
    lg_ref[...] = jnp.dot(enc.astype(jnp.bfloat16),
                          wd_ref[...].astype(jnp.bfloat16),
                          preferred_element_type=jnp.float32)


def kernel(x, wih_f, whh_f, b_f, wih_b, whh_b, b_b, wb, bias_b, wd,
           gumbel_noise):
    B, F, T = x.shape
    H = whh_f.shape[0]
    G = wb.shape[-1]
    C = wd.shape[-1]
    Bp = _round_up(max(B, 8), 8)
    Gp = _round_up(max(G, 128), 128)
    Cp = _round_up(max(C, 128), 128)
    TBp = T * Bp

    x_tbf = jnp.transpose(x.astype(jnp.bfloat16), (2, 0, 1))
    x_tbf = jnp.pad(x_tbf, ((0, 0), (0, Bp - B), (0, 0)))
    x_2d = x_tbf.reshape(TBp, F)

    lstm = functools.partial(_lstm_kernel, seq_len=T, batch=Bp, hidden=H)
    full2 = lambda i: (0, 0)
    embed_tm = pl.pallas_call(
        lstm,
        grid=(1,),
        out_shape=jax.ShapeDtypeStruct((TBp, 2 * H), jnp.bfloat16),
        in_specs=[
            pl.BlockSpec((TBp, F), full2),
            pl.BlockSpec((F, 4 * H), full2),
            pl.BlockSpec((H, 4 * H), full2),
            pl.BlockSpec((1, 4 * H), full2),
            pl.BlockSpec((F, 4 * H), full2),
            pl.BlockSpec((H, 4 * H), full2),
            pl.BlockSpec((1, 4 * H), full2),
        ],
        out_specs=pl.BlockSpec((TBp, 2 * H), full2),
        scratch_shapes=[pltpu.VMEM((TBp, 4 * H), jnp.float32),
                        pltpu.VMEM((TBp, 4 * H), jnp.float32)],
        compiler_params=pltpu.CompilerParams(
            dimension_semantics=("arbitrary",)),
    )(x_2d, wih_f, whh_f, b_f, wih_b, whh_b, b_b)

    e_3d = embed_tm.reshape(T, Bp, 2 * H)

    gum_2d = gumbel_noise.reshape(B * T, G)
    gum_2d = jnp.pad(gum_2d, ((0, (Bp - B) * T), (0, Gp - G)))

    wb_p = jnp.pad(wb, ((0, 0), (0, Gp - G)))
    bb_p = jnp.pad(bias_b, ((0, 0), (0, Gp - G)))
    wd_p = jnp.pad(wd, ((0, Gp - G), (0, Cp - C)))

    BT = 8
    NBLK = Bp // BT
    R = BT * T
    row_map = lambda j: (j, 0)
    wmap = lambda j: (0, 0)
    head = functools.partial(_head_kernel, inv_temp=1.0, n_gumbel=G, gp=Gp)
    il2, lg2 = pl.pallas_call(
        head,
        grid=(NBLK,),
        out_shape=(jax.ShapeDtypeStruct((TBp, Gp), jnp.float32),
                   jax.ShapeDtypeStruct((TBp, Cp), jnp.float32)),
        in_specs=[
            pl.BlockSpec((T, BT, 2 * H), lambda j: (0, j, 0)),
            pl.BlockSpec((R, Gp), row_map),
            pl.BlockSpec((2 * H, Gp), wmap),
            pl.BlockSpec((1, Gp), wmap),
            pl.BlockSpec((Gp, Cp), wmap),
        ],
        out_specs=(pl.BlockSpec((R, Gp), row_map),
                   pl.BlockSpec((R, Cp), row_map)),
        compiler_params=pltpu.CompilerParams(
            dimension_semantics=("arbitrary",)),
    )(e_3d, gum_2d, wb_p, bb_p, wd_p)

    in_logit = il2.reshape(Bp, T, Gp)[:B, :, :G]
    logit = lg2.reshape(Bp, T, Cp)[:B, :, :C]
    return in_logit, logit
